# Initial kernel scaffold; baseline (speedup 1.0000x reference)
import jax
import jax.numpy as jnp
from jax.experimental import pallas as pl


def kernel(choice, choice_for_updating, reward, alpha_plus, alpha_minus, initial_values, beta_info, beta_stick, beta_temp):
    raise NotImplementedError("write your pallas kernel here")


# speedup vs baseline: 6.5027x; 6.5027x over previous
import functools

import jax
import jax.numpy as jnp
from jax import lax
from jax.experimental import pallas as pl
from jax.experimental.pallas import tpu as pltpu
from jax.experimental.pallas import tpu_sc as plsc

N = 16384
T = 200
K = 3
NC = 2
NS = 16
L = 16
NW = NC * NS
RPW = N // NW
G = 2
SGR = G * L
SG = RPW // SGR


def _build():
    mesh = plsc.VectorSubcoreMesh(core_axis_name="c", subcore_axis_name="s")

    @functools.partial(
        pl.kernel,
        out_type=jax.ShapeDtypeStruct((N * T * K,), jnp.float32),
        mesh=mesh,
        compiler_params=pltpu.CompilerParams(needs_layout_passes=False),
        scratch_types=[
            pltpu.VMEM((SGR * T,), jnp.int32),
            pltpu.VMEM((SGR * T,), jnp.int32),
            pltpu.VMEM((SGR * T,), jnp.float32),
            pltpu.VMEM((SGR * T * K,), jnp.float32),
            pltpu.VMEM((8, L), jnp.float32),
        ],
    )
    def rw_kernel(choice_hbm, cfu_hbm, reward_hbm, params_hbm, out_hbm,
                  ch_v, cfu_v, rew_v, out_v, par_v):
        wid = lax.axis_index("s") * NC + lax.axis_index("c")

        pltpu.sync_copy(params_hbm, par_v)
        ap = par_v[0]
        am = par_v[1]
        init_v = par_v[2]
        bi = par_v[3]
        bs = par_v[4]
        bt = par_v[5]

        lane = lax.iota(jnp.int32, L)
        ibase = [lane * T + (g * L * T) for g in range(G)]
        obase = [lane * (T * K) + (g * L * T * K) for g in range(G)]
        one_i = jnp.full((L,), 1, jnp.int32)
        two_i = jnp.full((L,), 2, jnp.int32)
        three_i = jnp.full((L,), 3, jnp.int32)
        zero_i = jnp.zeros((L,), jnp.int32)
        zero_f = jnp.zeros((L,), jnp.float32)
        one_f = jnp.full((L,), 1.0, jnp.float32)

        def super_group(sg, acc):
            row0 = wid * RPW + sg * SGR
            pltpu.sync_copy(choice_hbm.at[pl.ds(row0 * T, SGR * T)], ch_v)
            pltpu.sync_copy(cfu_hbm.at[pl.ds(row0 * T, SGR * T)], cfu_v)
            pltpu.sync_copy(reward_hbm.at[pl.ds(row0 * T, SGR * T)], rew_v)

            def step(t, carry):
                tvec, ovec = carry[0], carry[1]
                st = list(carry[2:])
                new_st = []
                for g in range(G):
                    v0, v1, v2, c0, c1, c2, s0, s1, s2 = st[g * 9:(g + 1) * 9]
                    ig = ibase[g] + tvec
                    og = obase[g] + ovec
                    plsc.store_scatter(out_v, [og],
                                       v0 * bt + (c0 * bi + s0))
                    plsc.store_scatter(out_v, [og + one_i],
                                       v1 * bt + (c1 * bi + s1))
                    plsc.store_scatter(out_v, [og + two_i],
                                       v2 * bt + (c2 * bi + s2))
                    ch = plsc.load_gather(ch_v, [ig])
                    h0 = ch == zero_i
                    h1 = ch == one_i
                    h2 = ch == two_i
                    s0 = jnp.where(h0, bs, zero_f)
                    s1 = jnp.where(h1, bs, zero_f)
                    s2 = jnp.where(h2, bs, zero_f)
                    c0 = c0 + jnp.where(h0, one_f, zero_f)
                    c1 = c1 + jnp.where(h1, one_f, zero_f)
                    c2 = c2 + jnp.where(h2, one_f, zero_f)
                    c = plsc.load_gather(cfu_v, [ig])
                    r = plsc.load_gather(rew_v, [ig])
                    m0 = c == zero_i
                    m1 = c == one_i
                    m2 = c == two_i
                    chosen = jnp.where(m0, v0, jnp.where(m1, v1, v2))
                    pe = r - chosen
                    pe = jnp.where(r != r, zero_f, pe)
                    coef = jnp.where(pe >= zero_f, ap, am)
                    upd = chosen + coef * pe
                    v0 = jnp.where(m0, upd, v0)
                    v1 = jnp.where(m1, upd, v1)
                    v2 = jnp.where(m2, upd, v2)
                    new_st += [v0, v1, v2, c0, c1, c2, s0, s1, s2]
                return tuple([tvec + one_i, ovec + three_i] + new_st)

            init = [zero_i, zero_i]
            for g in range(G):
                init += [init_v, init_v, init_v,
                         zero_f, zero_f, zero_f,
                         zero_f, zero_f, zero_f]
            lax.fori_loop(0, T, step, tuple(init), unroll=False)

            pltpu.sync_copy(out_v, out_hbm.at[pl.ds(row0 * T * K, SGR * T * K)])
            return acc

        lax.fori_loop(0, SG, super_group, 0, unroll=False)

    return rw_kernel


def kernel(choice, choice_for_updating, reward, alpha_plus, alpha_minus,
           initial_values, beta_info, beta_stick, beta_temp):
    ap = jax.nn.sigmoid(alpha_plus)
    am = jax.nn.sigmoid(alpha_minus)
    init_v = 100.0 * jnp.tanh(initial_values)
    params = jnp.stack([ap, am, init_v, beta_info, beta_stick, beta_temp,
                        jnp.float32(0.0), jnp.float32(0.0)])
    params = jnp.broadcast_to(params[:, None], (8, L)).astype(jnp.float32)
    fn = _build()
    out = fn(choice.reshape(-1), choice_for_updating.reshape(-1),
             reward.reshape(-1), params)
    return out.reshape(N, T, K)
